# Initial kernel scaffold; baseline (speedup 1.0000x reference)
import jax
import jax.numpy as jnp
from jax.experimental import pallas as pl


def kernel(x, g_cg, pos_posw, Wm0, bm0, Wm1, bm1, Wc0, bc0, Wc1, bc1, Wn0, bn0, Wn1, bn1):
    raise NotImplementedError("write your pallas kernel here")


# speedup vs baseline: 5.3383x; 5.3383x over previous
import functools

import jax
import jax.numpy as jnp
from jax import lax
from jax.experimental import pallas as pl
from jax.experimental.pallas import tpu as pltpu
from jax.experimental.pallas import tpu_sc as plsc

NC = 2
NS = 16
SUB = 80
KSUB = 2
CG = SUB * KSUB


def _tc_tables(x, pos_posw, Wstack, Wpstack, bstack):
    N, L = x.shape
    Bn = 2000 if N % 2000 == 0 else N
    nb = N // Bn

    def body(x_ref, p_ref, w_ref, wp_ref, b_ref, o_ref):
        mm = jnp.dot(x_ref[...], w_ref[0], preferred_element_type=jnp.float32)
        mm = mm + jnp.dot(p_ref[...], wp_ref[0], preferred_element_type=jnp.float32)
        o_ref[...] = mm + b_ref[0]

    return pl.pallas_call(
        body,
        grid=(4, nb),
        in_specs=[
            pl.BlockSpec((Bn, L), lambda t, b: (b, 0)),
            pl.BlockSpec((Bn, 6), lambda t, b: (b, 0)),
            pl.BlockSpec((1, L, L), lambda t, b: (t, 0, 0)),
            pl.BlockSpec((1, 6, L), lambda t, b: (t, 0, 0)),
            pl.BlockSpec((1, 1, L), lambda t, b: (t, 0, 0)),
        ],
        out_specs=pl.BlockSpec((Bn, L), lambda t, b: (t * nb + b, 0)),
        out_shape=jax.ShapeDtypeStruct((4 * N, L), jnp.float32),
    )(x, pos_posw, Wstack, Wpstack, bstack.reshape(4, 1, L))


def _rsqrt16(x):
    xi = lax.bitcast_convert_type(x, jnp.int32)
    yi = jnp.int32(0x5F3759DF) - (xi >> 1)
    y = lax.bitcast_convert_type(yi, jnp.float32)
    for _ in range(3):
        y = y * (1.5 - 0.5 * x * y * y)
    return y


def _sc_gather(tab, ia3, ib3, pos_flat, wn, N):
    ntot = ia3.shape[0]
    L = tab.shape[1]
    EE = ntot * CG
    nchunk = ntot // (NC * NS)
    npos = pos_flat.shape[0]
    nsl = L // 16
    mesh = plsc.VectorSubcoreMesh(core_axis_name="c", subcore_axis_name="s")

    @functools.partial(
        pl.kernel,
        mesh=mesh,
        out_type=jax.ShapeDtypeStruct((EE, L), jnp.float32),
        scratch_types=[
            pltpu.VMEM((KSUB, SUB), jnp.int32),
            pltpu.VMEM((KSUB, SUB), jnp.int32),
            pltpu.VMEM((CG, L), jnp.float32),
            pltpu.VMEM((CG, L), jnp.float32),
            pltpu.VMEM((npos,), jnp.float32),
            pltpu.VMEM((4, L), jnp.float32),
            pltpu.SemaphoreType.DMA,
            pltpu.SemaphoreType.DMA,
        ],
        compiler_params=pltpu.CompilerParams(needs_layout_passes=False),
    )
    def gk(tab_h, ia_h, ib_h, pos_h, wn_h, out_h, ia_v, ib_v, ra_v, rb_v,
           pos_v, wn_v, sa, sb):
        wid = lax.axis_index("s") * NC + lax.axis_index("c")
        chunk0 = wid * nchunk
        pltpu.sync_copy(pos_h, pos_v)
        pltpu.sync_copy(wn_h, wn_v)
        is_main = wid < (NC * NS) // 2
        off_a = jnp.where(is_main, 0, 2 * N)
        off_b = jnp.where(is_main, N, 3 * N)
        toff = jnp.where(is_main, 0, 2)
        wnd = [wn_v[toff, pl.ds(k * 16, 16)] for k in range(nsl)]
        wndw = [wn_v[toff + 1, pl.ds(k * 16, 16)] for k in range(nsl)]

        def chunk(k, carry):
            cidx = chunk0 + k
            pltpu.sync_copy(ia_h.at[cidx], ia_v)
            pltpu.sync_copy(ib_h.at[cidx], ib_v)
            descs = []
            for b in range(KSUB):
                dst_a = ra_v.at[pl.ds(b * SUB, SUB)]
                dst_b = rb_v.at[pl.ds(b * SUB, SUB)]
                descs.append(pltpu.async_copy(tab_h.at[ia_v.at[b]], dst_a, sa))
                descs.append(pltpu.async_copy(tab_h.at[ib_v.at[b]], dst_b, sb))
            for d in descs:
                d.wait()

            for b in range(KSUB):

                def group(l, cc, b=b):
                    ia16 = ia_v[b, pl.ds(l * 16, 16)] - off_a
                    ib16 = ib_v[b, pl.ds(l * 16, 16)] - off_b
                    pa = ia16 * 6
                    pb = ib16 * 6
                    gi = [plsc.load_gather(pos_v, [pa + k]) for k in range(6)]
                    gj = [plsc.load_gather(pos_v, [pb + k]) for k in range(6)]
                    d0 = gi[0] - gj[0]
                    d1 = gi[1] - gj[1]
                    d2 = gi[2] - gj[2]
                    w0 = gi[3] - gj[3]
                    w1 = gi[4] - gj[4]
                    w2 = gi[5] - gj[5]
                    sd = d0 * d0 + d1 * d1 + d2 * d2
                    sw = w0 * w0 + w1 * w1 + w2 * w2
                    nd = sd * _rsqrt16(jnp.maximum(sd, 1e-30))
                    ndw = sw * _rsqrt16(jnp.maximum(sw, 1e-30))
                    row0 = b * SUB + l * 16
                    for e in range(16):
                        cnd = nd[e]
                        cndw = ndw[e]
                        r = row0 + e
                        for k in range(nsl):
                            sl = pl.ds(k * 16, 16)
                            acc = ra_v[r, sl] + rb_v[r, sl]
                            acc = acc + cnd * wnd[k] + cndw * wndw[k]
                            ra_v[r, sl] = acc
                    return cc

                lax.fori_loop(0, SUB // 16, group, 0)

            pltpu.sync_copy(ra_v, out_h.at[pl.ds(cidx * CG, CG)])
            return carry

        lax.fori_loop(0, nchunk, chunk, 0)

    return gk(tab, ia3, ib3, pos_flat, wn)


def _tc_edge(S, W1s, b1s, E):
    EE = S.shape[0]
    Be = 1000 if E % 1000 == 0 else E
    nbe = E // Be

    def body(s_ref, w1_ref, b1_ref, o_ref):
        h = jnp.maximum(s_ref[...], 0.0)
        h1 = jnp.dot(h, w1_ref[0], preferred_element_type=jnp.float32)
        h1 = h1 + b1_ref[0]
        mu = jnp.mean(h1, axis=1, keepdims=True)
        var = jnp.mean((h1 - mu) ** 2, axis=1, keepdims=True)
        o_ref[...] = (h1 - mu) * lax.rsqrt(var + 1e-5)

    return pl.pallas_call(
        body,
        grid=(EE // Be,),
        in_specs=[
            pl.BlockSpec((Be, 128), lambda t: (t, 0)),
            pl.BlockSpec((1, 128, 128), lambda t: (t // nbe, 0, 0)),
            pl.BlockSpec((1, 1, 128), lambda t: (t // nbe, 0, 0)),
        ],
        out_specs=pl.BlockSpec((Be, 128), lambda t: (t, 0)),
        out_shape=jax.ShapeDtypeStruct((EE, 128), jnp.float32),
    )(S, W1s, b1s.reshape(2, 1, 128))


def _sc_scatter(e2, j4, zeros, NP):
    _, E, L = e2.shape
    _, ntot, ks, _ = j4.shape
    C = ks * SUB
    nchunk = ntot // NS
    per_s = E // NS
    nrow = NP // NS
    mesh = plsc.VectorSubcoreMesh(core_axis_name="c", subcore_axis_name="s")

    @functools.partial(
        pl.kernel,
        mesh=mesh,
        out_type=jax.ShapeDtypeStruct((NC, NP, L), jnp.float32),
        scratch_types=[
            pltpu.VMEM((ks, SUB), jnp.int32),
            pltpu.VMEM((C, L), jnp.float32),
            pltpu.VMEM_SHARED((NP, L), jnp.float32),
            pltpu.SemaphoreType.DMA,
        ],
    )
    def sk(e_h, j_h, z_h, out_h, j_v, rows_v, acc, sem):
        c = lax.axis_index("c")
        s = lax.axis_index("s")
        pltpu.sync_copy(z_h.at[pl.ds(s * nrow, nrow)], acc.at[pl.ds(s * nrow, nrow)])
        plsc.subcore_barrier()

        def chunk(k, carry):
            cidx = s * nchunk + k
            pltpu.sync_copy(j_h.at[c, cidx], j_v)
            pltpu.sync_copy(e_h.at[c, pl.ds(s * per_s + k * C, C)], rows_v)
            for b in range(ks):
                src = rows_v.at[pl.ds(b * SUB, SUB)]
                pltpu.sync_copy(src, acc.at[j_v.at[b]], add=True)
            return carry

        lax.fori_loop(0, nchunk, chunk, 0)
        plsc.subcore_barrier()
        pltpu.sync_copy(acc.at[pl.ds(s * nrow, nrow)], out_h.at[c, pl.ds(s * nrow, nrow)])

    return sk(e2, j4, zeros)


def _tc_node(x, am, ac, Wn0, bn0, Wn1, bn1):
    N, L = x.shape
    Bn = 2000 if N % 2000 == 0 else N

    def body(x_ref, a_ref, c_ref, w0_ref, b0_ref, w1_ref, b1_ref, o_ref):
        xb = x_ref[...]
        h = jnp.dot(xb, w0_ref[0:128, :], preferred_element_type=jnp.float32)
        h = h + jnp.dot(a_ref[...], w0_ref[128:256, :], preferred_element_type=jnp.float32)
        h = h + jnp.dot(c_ref[...], w0_ref[256:384, :], preferred_element_type=jnp.float32)
        h = jnp.maximum(h + b0_ref[0], 0.0)
        h1 = jnp.dot(h, w1_ref[...], preferred_element_type=jnp.float32)
        h1 = h1 + b1_ref[0]
        mu = jnp.mean(h1, axis=1, keepdims=True)
        var = jnp.mean((h1 - mu) ** 2, axis=1, keepdims=True)
        o_ref[...] = (h1 - mu) * lax.rsqrt(var + 1e-5) + xb

    return pl.pallas_call(
        body,
        grid=(N // Bn,),
        in_specs=[
            pl.BlockSpec((Bn, L), lambda b: (b, 0)),
            pl.BlockSpec((Bn, L), lambda b: (b, 0)),
            pl.BlockSpec((Bn, L), lambda b: (b, 0)),
            pl.BlockSpec((384, L), lambda b: (0, 0)),
            pl.BlockSpec((1, L), lambda b: (0, 0)),
            pl.BlockSpec((L, L), lambda b: (0, 0)),
            pl.BlockSpec((1, L), lambda b: (0, 0)),
        ],
        out_specs=pl.BlockSpec((Bn, L), lambda b: (b, 0)),
        out_shape=jax.ShapeDtypeStruct((N, L), jnp.float32),
    )(x, am, ac, Wn0, bn0.reshape(1, L), Wn1, bn1.reshape(1, L))


def kernel(x, g_cg, pos_posw, Wm0, bm0, Wm1, bm1, Wc0, bc0, Wc1, bc1,
           Wn0, bn0, Wn1, bn1):
    N, L = x.shape
    E = g_cg.shape[-1]
    z3 = jnp.zeros((3, L), jnp.float32)

    Wstack = jnp.stack([Wm0[8:136], Wm0[136:264], Wc0[4:132], Wc0[132:260]])
    Wp_m = jnp.concatenate([Wm0[0:3], Wm0[4:7]])
    Wp_c = jnp.concatenate([z3, Wc0[0:3]])
    Wpstack = jnp.stack([Wp_m, -Wp_m, Wp_c, -Wp_c])
    zb = jnp.zeros_like(bm0)
    bstack = jnp.stack([bm0, zb, bc0, zb])
    tab = _tc_tables(x, pos_posw, Wstack, Wpstack, bstack)

    wn = jnp.stack([Wm0[3], Wm0[7], jnp.zeros((L,), jnp.float32), Wc0[3]])

    iA = jnp.concatenate([g_cg[0, 0], g_cg[1, 0] + 2 * N])
    iB = jnp.concatenate([g_cg[0, 1] + N, g_cg[1, 1] + 3 * N])
    npos = ((6 * N + 127) // 128) * 128
    pos_flat = jnp.zeros((npos,), jnp.float32).at[: 6 * N].set(pos_posw.reshape(-1))
    S = _sc_gather(tab, iA.reshape(-1, KSUB, SUB), iB.reshape(-1, KSUB, SUB),
                   pos_flat, wn, N)

    W1s = jnp.stack([Wm1, Wc1])
    b1s = jnp.stack([bm1, bc1])
    e = _tc_edge(S, W1s, b1s, E)

    NP = ((N + 8 * NS - 1) // (8 * NS)) * (8 * NS)
    j4 = g_cg[:, 1, :].reshape(2, E // CG, KSUB, SUB)
    agg = _sc_scatter(e.reshape(2, E, L), j4, jnp.zeros((NP, L), jnp.float32), NP)

    return _tc_node(x, agg[0, :N], agg[1, :N], Wn0, bn0, Wn1, bn1)


# speedup vs baseline: 8.3762x; 1.5691x over previous
import functools

import jax
import jax.numpy as jnp
from jax import lax
from jax.experimental import pallas as pl
from jax.experimental.pallas import tpu as pltpu
from jax.experimental.pallas import tpu_sc as plsc

NC = 2
NS = 16
SUB = 80
CG = SUB
KS2 = 1
CS = SUB * KS2


def _tc_tables(x, pos_posw, Wstack, Wpstack, bstack):
    N, L = x.shape
    Bn = 2000 if N % 2000 == 0 else N
    nb = N // Bn

    def body(x_ref, p_ref, w_ref, wp_ref, b_ref, o_ref):
        mm = jnp.dot(x_ref[...], w_ref[0], preferred_element_type=jnp.float32)
        mm = mm + jnp.dot(p_ref[...], wp_ref[0], preferred_element_type=jnp.float32)
        o_ref[...] = mm + b_ref[0]

    return pl.pallas_call(
        body,
        grid=(4, nb),
        in_specs=[
            pl.BlockSpec((Bn, L), lambda t, b: (b, 0)),
            pl.BlockSpec((Bn, 6), lambda t, b: (b, 0)),
            pl.BlockSpec((1, L, L), lambda t, b: (t, 0, 0)),
            pl.BlockSpec((1, 6, L), lambda t, b: (t, 0, 0)),
            pl.BlockSpec((1, 1, L), lambda t, b: (t, 0, 0)),
        ],
        out_specs=pl.BlockSpec((Bn, L), lambda t, b: (t * nb + b, 0)),
        out_shape=jax.ShapeDtypeStruct((4 * N, L), jnp.float32),
    )(x, pos_posw, Wstack, Wpstack, bstack.reshape(4, 1, L))


def _rsqrt16(x):
    xi = lax.bitcast_convert_type(x, jnp.int32)
    yi = jnp.int32(0x5F3759DF) - (xi >> 1)
    y = lax.bitcast_convert_type(yi, jnp.float32)
    for _ in range(3):
        y = y * (1.5 - 0.5 * x * y * y)
    return y


def _sc_gather(tab, ia3, ib3, pos_flat, wn, N):
    ntot = ia3.shape[0]
    L = tab.shape[1]
    EE = ntot * CG
    nchunk = ntot // (NC * NS)
    npos = pos_flat.shape[0]
    nsl = L // 16
    mesh = plsc.VectorSubcoreMesh(core_axis_name="c", subcore_axis_name="s")

    @functools.partial(
        pl.kernel,
        mesh=mesh,
        out_type=jax.ShapeDtypeStruct((EE, L), jnp.float32),
        scratch_types=[
            pltpu.VMEM((2, 1, SUB), jnp.int32),
            pltpu.VMEM((2, 1, SUB), jnp.int32),
            pltpu.VMEM((2, CG, L), jnp.float32),
            pltpu.VMEM((2, CG, L), jnp.float32),
            pltpu.VMEM((npos,), jnp.float32),
            pltpu.VMEM((4, L), jnp.float32),
            [pltpu.SemaphoreType.DMA] * 2,
            [pltpu.SemaphoreType.DMA] * 2,
            [pltpu.SemaphoreType.DMA] * 2,
        ],
        compiler_params=pltpu.CompilerParams(needs_layout_passes=False),
    )
    def gk(tab_h, ia_h, ib_h, pos_h, wn_h, out_h, ia_v, ib_v, ra_v, rb_v,
           pos_v, wn_v, si, sg, sw):
        wid = lax.axis_index("s") * NC + lax.axis_index("c")
        chunk0 = wid * nchunk
        pltpu.sync_copy(pos_h, pos_v)
        pltpu.sync_copy(wn_h, wn_v)
        is_main = wid < (NC * NS) // 2
        off_a = jnp.where(is_main, 0, 2 * N)
        off_b = jnp.where(is_main, N, 3 * N)
        toff = jnp.where(is_main, 0, 2)
        wnd = [wn_v[toff, pl.ds(k * 16, 16)] for k in range(nsl)]
        wndw = [wn_v[toff + 1, pl.ds(k * 16, 16)] for k in range(nsl)]

        def issue_idx(cidx, d):
            pltpu.async_copy(ia_h.at[cidx], ia_v.at[d], si[d])
            pltpu.async_copy(ib_h.at[cidx], ib_v.at[d], si[d])

        def wait_idx(d):
            pltpu.make_async_copy(ia_h.at[0], ia_v.at[d], si[d]).wait()
            pltpu.make_async_copy(ib_h.at[0], ib_v.at[d], si[d]).wait()

        def issue_gather(d):
            pltpu.async_copy(tab_h.at[ia_v.at[d, 0]], ra_v.at[d], sg[d])
            pltpu.async_copy(tab_h.at[ib_v.at[d, 0]], rb_v.at[d], sg[d])

        def wait_gather(d):
            pltpu.make_async_copy(tab_h.at[ia_v.at[d, 0]], ra_v.at[d], sg[d]).wait()
            pltpu.make_async_copy(tab_h.at[ib_v.at[d, 0]], rb_v.at[d], sg[d]).wait()

        def issue_write(cidx, d):
            pltpu.async_copy(ra_v.at[d], out_h.at[pl.ds(cidx * CG, CG)], sw[d])

        def wait_write(d):
            pltpu.make_async_copy(ra_v.at[d], out_h.at[pl.ds(0, CG)], sw[d]).wait()

        def compute(d):
            def group(l, cc):
                ia16 = ia_v[d, 0, pl.ds(l * 16, 16)] - off_a
                ib16 = ib_v[d, 0, pl.ds(l * 16, 16)] - off_b
                pa = ia16 * 6
                pb = ib16 * 6
                gi = [plsc.load_gather(pos_v, [pa + k]) for k in range(6)]
                gj = [plsc.load_gather(pos_v, [pb + k]) for k in range(6)]
                d0 = gi[0] - gj[0]
                d1 = gi[1] - gj[1]
                d2 = gi[2] - gj[2]
                w0 = gi[3] - gj[3]
                w1 = gi[4] - gj[4]
                w2 = gi[5] - gj[5]
                sd = d0 * d0 + d1 * d1 + d2 * d2
                sq = w0 * w0 + w1 * w1 + w2 * w2
                nd = sd * _rsqrt16(jnp.maximum(sd, 1e-30))
                ndw = sq * _rsqrt16(jnp.maximum(sq, 1e-30))
                for e in range(16):
                    cnd = nd[e]
                    cndw = ndw[e]
                    r = l * 16 + e
                    for k in range(nsl):
                        sl = pl.ds(k * 16, 16)
                        acc = ra_v[d, r, sl] + rb_v[d, r, sl]
                        acc = acc + cnd * wnd[k] + cndw * wndw[k]
                        ra_v[d, r, sl] = acc
                return cc

            lax.fori_loop(0, CG // 16, group, 0)

        issue_idx(chunk0, 0)
        wait_idx(0)
        issue_gather(0)
        issue_idx(chunk0 + 1, 1)

        def pair(k2, carry):
            for d in (0, 1):
                k = k2 * 2 + d
                nxt = 1 - d

                @pl.when(k + 1 < nchunk)
                def _():
                    wait_idx(nxt)

                    @pl.when(k > 0)
                    def _():
                        wait_write(nxt)

                    issue_gather(nxt)

                wait_gather(d)
                compute(d)
                issue_write(chunk0 + k, d)

                @pl.when(k + 2 < nchunk)
                def _():
                    issue_idx(chunk0 + k + 2, d)
            return carry

        lax.fori_loop(0, nchunk // 2, pair, 0)
        wait_write(0)
        wait_write(1)

    return gk(tab, ia3, ib3, pos_flat, wn)


def _tc_edge(S, W1s, b1s, E):
    EE = S.shape[0]
    Be = 2000 if E % 2000 == 0 else E
    nbe = E // Be

    def body(s_ref, w1_ref, b1_ref, o_ref):
        h = jnp.maximum(s_ref[...], 0.0)
        h1 = jnp.dot(h, w1_ref[0], preferred_element_type=jnp.float32)
        h1 = h1 + b1_ref[0]
        mu = jnp.mean(h1, axis=1, keepdims=True)
        var = jnp.mean((h1 - mu) ** 2, axis=1, keepdims=True)
        o_ref[...] = (h1 - mu) * lax.rsqrt(var + 1e-5)

    return pl.pallas_call(
        body,
        grid=(EE // Be,),
        in_specs=[
            pl.BlockSpec((Be, 128), lambda t: (t, 0)),
            pl.BlockSpec((1, 128, 128), lambda t: (t // nbe, 0, 0)),
            pl.BlockSpec((1, 1, 128), lambda t: (t // nbe, 0, 0)),
        ],
        out_specs=pl.BlockSpec((Be, 128), lambda t: (t, 0)),
        out_shape=jax.ShapeDtypeStruct((EE, 128), jnp.float32),
    )(S, W1s, b1s.reshape(2, 1, 128))


def _sc_scatter(e2, j4, zeros, NP):
    _, E, L = e2.shape
    _, ntot, ks, _ = j4.shape
    C = ks * SUB
    nchunk = ntot // NS
    per_s = E // NS
    nrow = NP // NS
    mesh = plsc.VectorSubcoreMesh(core_axis_name="c", subcore_axis_name="s")

    @functools.partial(
        pl.kernel,
        mesh=mesh,
        out_type=jax.ShapeDtypeStruct((NC, NP, L), jnp.float32),
        scratch_types=[
            pltpu.VMEM((2, ks, SUB), jnp.int32),
            pltpu.VMEM((2, C, L), jnp.float32),
            pltpu.VMEM_SHARED((NP, L), jnp.float32),
            [pltpu.SemaphoreType.DMA] * 2,
            [pltpu.SemaphoreType.DMA] * 2,
        ],
    )
    def sk(e_h, j_h, z_h, out_h, j_v, rows_v, acc, sl, sa):
        c = lax.axis_index("c")
        s = lax.axis_index("s")
        pltpu.sync_copy(z_h.at[pl.ds(s * nrow, nrow)], acc.at[pl.ds(s * nrow, nrow)])
        plsc.subcore_barrier()

        def issue_load(k, d):
            cidx = s * nchunk + k
            pltpu.async_copy(j_h.at[c, cidx], j_v.at[d], sl[d])
            pltpu.async_copy(e_h.at[c, pl.ds(s * per_s + k * C, C)], rows_v.at[d], sl[d])

        def wait_load(d):
            pltpu.make_async_copy(j_h.at[c, 0], j_v.at[d], sl[d]).wait()
            pltpu.make_async_copy(e_h.at[c, pl.ds(0, C)], rows_v.at[d], sl[d]).wait()

        def issue_adds(d):
            for b in range(ks):
                src = rows_v.at[d, pl.ds(b * SUB, SUB)]
                pltpu.async_copy(src, acc.at[j_v.at[d, b]], sa[d], add=True)

        def wait_adds(d):
            for b in range(ks):
                src = rows_v.at[d, pl.ds(b * SUB, SUB)]
                pltpu.make_async_copy(src, acc.at[j_v.at[d, b]], sa[d]).wait()

        issue_load(0, 0)
        issue_load(1, 1)

        def pair(k2, carry):
            for d in (0, 1):
                k = k2 * 2 + d
                wait_load(d)
                issue_adds(d)

                @pl.when(k + 2 < nchunk)
                def _():
                    wait_adds(d)
                    issue_load(k + 2, d)
            return carry

        lax.fori_loop(0, nchunk // 2, pair, 0)
        wait_adds(0)
        wait_adds(1)
        plsc.subcore_barrier()
        pltpu.sync_copy(acc.at[pl.ds(s * nrow, nrow)], out_h.at[c, pl.ds(s * nrow, nrow)])

    return sk(e2, j4, zeros)


def _tc_node(x, am, ac, Wn0, bn0, Wn1, bn1):
    N, L = x.shape
    Bn = 2000 if N % 2000 == 0 else N

    def body(x_ref, a_ref, c_ref, w0_ref, b0_ref, w1_ref, b1_ref, o_ref):
        xb = x_ref[...]
        h = jnp.dot(xb, w0_ref[0:128, :], preferred_element_type=jnp.float32)
        h = h + jnp.dot(a_ref[...], w0_ref[128:256, :], preferred_element_type=jnp.float32)
        h = h + jnp.dot(c_ref[...], w0_ref[256:384, :], preferred_element_type=jnp.float32)
        h = jnp.maximum(h + b0_ref[0], 0.0)
        h1 = jnp.dot(h, w1_ref[...], preferred_element_type=jnp.float32)
        h1 = h1 + b1_ref[0]
        mu = jnp.mean(h1, axis=1, keepdims=True)
        var = jnp.mean((h1 - mu) ** 2, axis=1, keepdims=True)
        o_ref[...] = (h1 - mu) * lax.rsqrt(var + 1e-5) + xb

    return pl.pallas_call(
        body,
        grid=(N // Bn,),
        in_specs=[
            pl.BlockSpec((Bn, L), lambda b: (b, 0)),
            pl.BlockSpec((Bn, L), lambda b: (b, 0)),
            pl.BlockSpec((Bn, L), lambda b: (b, 0)),
            pl.BlockSpec((384, L), lambda b: (0, 0)),
            pl.BlockSpec((1, L), lambda b: (0, 0)),
            pl.BlockSpec((L, L), lambda b: (0, 0)),
            pl.BlockSpec((1, L), lambda b: (0, 0)),
        ],
        out_specs=pl.BlockSpec((Bn, L), lambda b: (b, 0)),
        out_shape=jax.ShapeDtypeStruct((N, L), jnp.float32),
    )(x, am, ac, Wn0, bn0.reshape(1, L), Wn1, bn1.reshape(1, L))


def kernel(x, g_cg, pos_posw, Wm0, bm0, Wm1, bm1, Wc0, bc0, Wc1, bc1,
           Wn0, bn0, Wn1, bn1):
    N, L = x.shape
    E = g_cg.shape[-1]
    z3 = jnp.zeros((3, L), jnp.float32)

    Wstack = jnp.stack([Wm0[8:136], Wm0[136:264], Wc0[4:132], Wc0[132:260]])
    Wp_m = jnp.concatenate([Wm0[0:3], Wm0[4:7]])
    Wp_c = jnp.concatenate([z3, Wc0[0:3]])
    Wpstack = jnp.stack([Wp_m, -Wp_m, Wp_c, -Wp_c])
    zb = jnp.zeros_like(bm0)
    bstack = jnp.stack([bm0, zb, bc0, zb])
    tab = _tc_tables(x, pos_posw, Wstack, Wpstack, bstack)

    wn = jnp.stack([Wm0[3], Wm0[7], jnp.zeros((L,), jnp.float32), Wc0[3]])

    iA = jnp.concatenate([g_cg[0, 0], g_cg[1, 0] + 2 * N])
    iB = jnp.concatenate([g_cg[0, 1] + N, g_cg[1, 1] + 3 * N])
    npos = ((6 * N + 127) // 128) * 128
    pos_flat = jnp.zeros((npos,), jnp.float32).at[: 6 * N].set(pos_posw.reshape(-1))
    S = _sc_gather(tab, iA.reshape(-1, 1, SUB), iB.reshape(-1, 1, SUB),
                   pos_flat, wn, N)

    W1s = jnp.stack([Wm1, Wc1])
    b1s = jnp.stack([bm1, bc1])
    e = _tc_edge(S, W1s, b1s, E)

    NP = ((N + 8 * NS - 1) // (8 * NS)) * (8 * NS)
    j4 = g_cg[:, 1, :].reshape(2, E // CS, KS2, SUB)
    agg = _sc_scatter(e.reshape(2, E, L), j4, jnp.zeros((NP, L), jnp.float32), NP)

    return _tc_node(x, agg[0, :N], agg[1, :N], Wn0, bn0, Wn1, bn1)


# speedup vs baseline: 8.4067x; 1.0036x over previous
import functools

import jax
import jax.numpy as jnp
from jax import lax
from jax.experimental import pallas as pl
from jax.experimental.pallas import tpu as pltpu
from jax.experimental.pallas import tpu_sc as plsc

NC = 2
NS = 16
SUB = 80
CG = SUB
KS2 = 1
CS = SUB * KS2


def _tc_tables(x, pos_posw, Wstack, Wpstack, bstack):
    N, L = x.shape
    Bn = 2000 if N % 2000 == 0 else N
    nb = N // Bn

    def body(x_ref, p_ref, w_ref, wp_ref, b_ref, o_ref):
        mm = jnp.dot(x_ref[...], w_ref[0], preferred_element_type=jnp.float32)
        mm = mm + jnp.dot(p_ref[...], wp_ref[0], preferred_element_type=jnp.float32)
        o_ref[...] = mm + b_ref[0]

    return pl.pallas_call(
        body,
        grid=(4, nb),
        in_specs=[
            pl.BlockSpec((Bn, L), lambda t, b: (b, 0)),
            pl.BlockSpec((Bn, 6), lambda t, b: (b, 0)),
            pl.BlockSpec((1, L, L), lambda t, b: (t, 0, 0)),
            pl.BlockSpec((1, 6, L), lambda t, b: (t, 0, 0)),
            pl.BlockSpec((1, 1, L), lambda t, b: (t, 0, 0)),
        ],
        out_specs=pl.BlockSpec((Bn, L), lambda t, b: (t * nb + b, 0)),
        out_shape=jax.ShapeDtypeStruct((4 * N, L), jnp.float32),
    )(x, pos_posw, Wstack, Wpstack, bstack.reshape(4, 1, L))


def _rsqrt16(x):
    xi = lax.bitcast_convert_type(x, jnp.int32)
    yi = jnp.int32(0x5F3759DF) - (xi >> 1)
    y = lax.bitcast_convert_type(yi, jnp.float32)
    for _ in range(3):
        y = y * (1.5 - 0.5 * x * y * y)
    return y


def _sc_gather(tab, ia3, ib3, pos_flat, wn, N):
    ntot = ia3.shape[0]
    L = tab.shape[1]
    EE = ntot * CG
    nchunk = ntot // (NC * NS)
    npos = pos_flat.shape[0]
    nsl = L // 16
    mesh = plsc.VectorSubcoreMesh(core_axis_name="c", subcore_axis_name="s")

    @functools.partial(
        pl.kernel,
        mesh=mesh,
        out_type=jax.ShapeDtypeStruct((EE, L), jnp.float32),
        scratch_types=[
            pltpu.VMEM((2, 1, SUB), jnp.int32),
            pltpu.VMEM((2, 1, SUB), jnp.int32),
            pltpu.VMEM((2, CG, L), jnp.float32),
            pltpu.VMEM((2, CG, L), jnp.float32),
            pltpu.VMEM((npos,), jnp.float32),
            pltpu.VMEM((4, L), jnp.float32),
            [pltpu.SemaphoreType.DMA] * 2,
            [pltpu.SemaphoreType.DMA] * 2,
            [pltpu.SemaphoreType.DMA] * 2,
        ],
        compiler_params=pltpu.CompilerParams(needs_layout_passes=False),
    )
    def gk(tab_h, ia_h, ib_h, pos_h, wn_h, out_h, ia_v, ib_v, ra_v, rb_v,
           pos_v, wn_v, si, sg, sw):
        wid = lax.axis_index("s") * NC + lax.axis_index("c")
        chunk0 = wid * nchunk
        pltpu.sync_copy(pos_h, pos_v)
        pltpu.sync_copy(wn_h, wn_v)
        is_main = wid < (NC * NS) // 2
        off_a = jnp.where(is_main, 0, 2 * N)
        off_b = jnp.where(is_main, N, 3 * N)
        toff = jnp.where(is_main, 0, 2)
        wnd = [wn_v[toff, pl.ds(k * 16, 16)] for k in range(nsl)]
        wndw = [wn_v[toff + 1, pl.ds(k * 16, 16)] for k in range(nsl)]

        def issue_idx(cidx, d):
            pltpu.async_copy(ia_h.at[cidx], ia_v.at[d], si[d])
            pltpu.async_copy(ib_h.at[cidx], ib_v.at[d], si[d])

        def wait_idx(d):
            pltpu.make_async_copy(ia_h.at[0], ia_v.at[d], si[d]).wait()
            pltpu.make_async_copy(ib_h.at[0], ib_v.at[d], si[d]).wait()

        def issue_gather(d):
            pltpu.async_copy(tab_h.at[ia_v.at[d, 0]], ra_v.at[d], sg[d])
            pltpu.async_copy(tab_h.at[ib_v.at[d, 0]], rb_v.at[d], sg[d])

        def wait_gather(d):
            pltpu.make_async_copy(tab_h.at[ia_v.at[d, 0]], ra_v.at[d], sg[d]).wait()
            pltpu.make_async_copy(tab_h.at[ib_v.at[d, 0]], rb_v.at[d], sg[d]).wait()

        def issue_write(cidx, d):
            pltpu.async_copy(ra_v.at[d], out_h.at[pl.ds(cidx * CG, CG)], sw[d])

        def wait_write(d):
            pltpu.make_async_copy(ra_v.at[d], out_h.at[pl.ds(0, CG)], sw[d]).wait()

        def compute(d):
            def group(l, cc):
                ia16 = ia_v[d, 0, pl.ds(l * 16, 16)] - off_a
                ib16 = ib_v[d, 0, pl.ds(l * 16, 16)] - off_b
                pa = ia16 * 6
                pb = ib16 * 6
                gi = [plsc.load_gather(pos_v, [pa + k]) for k in range(6)]
                gj = [plsc.load_gather(pos_v, [pb + k]) for k in range(6)]
                d0 = gi[0] - gj[0]
                d1 = gi[1] - gj[1]
                d2 = gi[2] - gj[2]
                w0 = gi[3] - gj[3]
                w1 = gi[4] - gj[4]
                w2 = gi[5] - gj[5]
                sd = d0 * d0 + d1 * d1 + d2 * d2
                sq = w0 * w0 + w1 * w1 + w2 * w2
                nd = sd * _rsqrt16(jnp.maximum(sd, 1e-30))
                ndw = sq * _rsqrt16(jnp.maximum(sq, 1e-30))
                for e in range(16):
                    cnd = nd[e]
                    cndw = ndw[e]
                    r = l * 16 + e
                    for k in range(nsl):
                        sl = pl.ds(k * 16, 16)
                        acc = ra_v[d, r, sl] + rb_v[d, r, sl]
                        ra_v[d, r, sl] = acc + cnd * wnd[k] + cndw * wndw[k]
                return cc

            lax.fori_loop(0, CG // 16, group, 0)

        issue_idx(chunk0, 0)
        wait_idx(0)
        issue_gather(0)
        issue_idx(chunk0 + 1, 1)

        def pair(k2, carry):
            for d in (0, 1):
                k = k2 * 2 + d
                nxt = 1 - d

                @pl.when(k + 1 < nchunk)
                def _():
                    wait_idx(nxt)

                    @pl.when(k > 0)
                    def _():
                        wait_write(nxt)

                    issue_gather(nxt)

                wait_gather(d)
                compute(d)
                issue_write(chunk0 + k, d)

                @pl.when(k + 2 < nchunk)
                def _():
                    issue_idx(chunk0 + k + 2, d)
            return carry

        lax.fori_loop(0, nchunk // 2, pair, 0)
        wait_write(0)
        wait_write(1)

    return gk(tab, ia3, ib3, pos_flat, wn)


def _tc_edge(S, W1s, b1s, E):
    EE = S.shape[0]
    Be = 2000 if E % 2000 == 0 else E
    nbe = E // Be

    def body(s_ref, w1_ref, b1_ref, o_ref):
        h = jnp.maximum(s_ref[...], 0.0).astype(jnp.bfloat16)
        h1 = jnp.dot(h, w1_ref[0], preferred_element_type=jnp.float32)
        h1 = h1 + b1_ref[0]
        mu = jnp.mean(h1, axis=1, keepdims=True)
        var = jnp.mean((h1 - mu) ** 2, axis=1, keepdims=True)
        o_ref[...] = (h1 - mu) * lax.rsqrt(var + 1e-5)

    return pl.pallas_call(
        body,
        grid=(EE // Be,),
        in_specs=[
            pl.BlockSpec((Be, 128), lambda t: (t, 0)),
            pl.BlockSpec((1, 128, 128), lambda t: (t // nbe, 0, 0)),
            pl.BlockSpec((1, 1, 128), lambda t: (t // nbe, 0, 0)),
        ],
        out_specs=pl.BlockSpec((Be, 128), lambda t: (t, 0)),
        out_shape=jax.ShapeDtypeStruct((EE, 128), jnp.float32),
    )(S, W1s, b1s.reshape(2, 1, 128))


def _sc_scatter(e2, j4, zeros, NP):
    _, E, L = e2.shape
    _, ntot, ks, _ = j4.shape
    C = ks * SUB
    nchunk = ntot // NS
    per_s = E // NS
    nrow = NP // NS
    mesh = plsc.VectorSubcoreMesh(core_axis_name="c", subcore_axis_name="s")

    @functools.partial(
        pl.kernel,
        mesh=mesh,
        out_type=jax.ShapeDtypeStruct((NC, NP, L), jnp.float32),
        scratch_types=[
            pltpu.VMEM((2, ks, SUB), jnp.int32),
            pltpu.VMEM((2, C, L), jnp.float32),
            pltpu.VMEM_SHARED((NP, L), jnp.float32),
            [pltpu.SemaphoreType.DMA] * 2,
            [pltpu.SemaphoreType.DMA] * 2,
        ],
    )
    def sk(e_h, j_h, z_h, out_h, j_v, rows_v, acc, sl, sa):
        c = lax.axis_index("c")
        s = lax.axis_index("s")
        pltpu.sync_copy(z_h.at[pl.ds(s * nrow, nrow)], acc.at[pl.ds(s * nrow, nrow)])
        plsc.subcore_barrier()

        def issue_load(k, d):
            cidx = s * nchunk + k
            pltpu.async_copy(j_h.at[c, cidx], j_v.at[d], sl[d])
            pltpu.async_copy(e_h.at[c, pl.ds(s * per_s + k * C, C)], rows_v.at[d], sl[d])

        def wait_load(d):
            pltpu.make_async_copy(j_h.at[c, 0], j_v.at[d], sl[d]).wait()
            pltpu.make_async_copy(e_h.at[c, pl.ds(0, C)], rows_v.at[d], sl[d]).wait()

        def issue_adds(d):
            for b in range(ks):
                src = rows_v.at[d, pl.ds(b * SUB, SUB)]
                pltpu.async_copy(src, acc.at[j_v.at[d, b]], sa[d], add=True)

        def wait_adds(d):
            for b in range(ks):
                src = rows_v.at[d, pl.ds(b * SUB, SUB)]
                pltpu.make_async_copy(src, acc.at[j_v.at[d, b]], sa[d]).wait()

        issue_load(0, 0)
        issue_load(1, 1)

        def pair(k2, carry):
            for d in (0, 1):
                k = k2 * 2 + d
                wait_load(d)
                issue_adds(d)

                @pl.when(k + 2 < nchunk)
                def _():
                    wait_adds(d)
                    issue_load(k + 2, d)
            return carry

        lax.fori_loop(0, nchunk // 2, pair, 0)
        wait_adds(0)
        wait_adds(1)
        plsc.subcore_barrier()
        pltpu.sync_copy(acc.at[pl.ds(s * nrow, nrow)], out_h.at[c, pl.ds(s * nrow, nrow)])

    return sk(e2, j4, zeros)


def _tc_node(x, am, ac, Wn0, bn0, Wn1, bn1):
    N, L = x.shape
    Bn = 2000 if N % 2000 == 0 else N

    def body(x_ref, a_ref, c_ref, w0_ref, b0_ref, w1_ref, b1_ref, o_ref):
        xb = x_ref[...]
        h = jnp.dot(xb, w0_ref[0:128, :], preferred_element_type=jnp.float32)
        h = h + jnp.dot(a_ref[...], w0_ref[128:256, :], preferred_element_type=jnp.float32)
        h = h + jnp.dot(c_ref[...], w0_ref[256:384, :], preferred_element_type=jnp.float32)
        h = jnp.maximum(h + b0_ref[0], 0.0)
        h1 = jnp.dot(h, w1_ref[...], preferred_element_type=jnp.float32)
        h1 = h1 + b1_ref[0]
        mu = jnp.mean(h1, axis=1, keepdims=True)
        var = jnp.mean((h1 - mu) ** 2, axis=1, keepdims=True)
        o_ref[...] = (h1 - mu) * lax.rsqrt(var + 1e-5) + xb

    return pl.pallas_call(
        body,
        grid=(N // Bn,),
        in_specs=[
            pl.BlockSpec((Bn, L), lambda b: (b, 0)),
            pl.BlockSpec((Bn, L), lambda b: (b, 0)),
            pl.BlockSpec((Bn, L), lambda b: (b, 0)),
            pl.BlockSpec((384, L), lambda b: (0, 0)),
            pl.BlockSpec((1, L), lambda b: (0, 0)),
            pl.BlockSpec((L, L), lambda b: (0, 0)),
            pl.BlockSpec((1, L), lambda b: (0, 0)),
        ],
        out_specs=pl.BlockSpec((Bn, L), lambda b: (b, 0)),
        out_shape=jax.ShapeDtypeStruct((N, L), jnp.float32),
    )(x, am, ac, Wn0, bn0.reshape(1, L), Wn1, bn1.reshape(1, L))


def kernel(x, g_cg, pos_posw, Wm0, bm0, Wm1, bm1, Wc0, bc0, Wc1, bc1,
           Wn0, bn0, Wn1, bn1):
    N, L = x.shape
    E = g_cg.shape[-1]
    z3 = jnp.zeros((3, L), jnp.float32)

    Wstack = jnp.stack([Wm0[8:136], Wm0[136:264], Wc0[4:132], Wc0[132:260]])
    Wp_m = jnp.concatenate([Wm0[0:3], Wm0[4:7]])
    Wp_c = jnp.concatenate([z3, Wc0[0:3]])
    Wpstack = jnp.stack([Wp_m, -Wp_m, Wp_c, -Wp_c])
    zb = jnp.zeros_like(bm0)
    bstack = jnp.stack([bm0, zb, bc0, zb])
    tab = _tc_tables(x, pos_posw, Wstack, Wpstack, bstack)

    wn = jnp.stack([Wm0[3], Wm0[7], jnp.zeros((L,), jnp.float32), Wc0[3]])

    iA = jnp.concatenate([g_cg[0, 0], g_cg[1, 0] + 2 * N])
    iB = jnp.concatenate([g_cg[0, 1] + N, g_cg[1, 1] + 3 * N])
    npos = ((6 * N + 127) // 128) * 128
    pos_flat = jnp.zeros((npos,), jnp.float32).at[: 6 * N].set(pos_posw.reshape(-1))
    S = _sc_gather(tab, iA.reshape(-1, 1, SUB), iB.reshape(-1, 1, SUB),
                   pos_flat, wn, N)

    W1s = jnp.stack([Wm1, Wc1]).astype(jnp.bfloat16)
    b1s = jnp.stack([bm1, bc1])
    e = _tc_edge(S, W1s, b1s, E)

    NP = ((N + 8 * NS - 1) // (8 * NS)) * (8 * NS)
    j4 = g_cg[:, 1, :].reshape(2, E // CS, KS2, SUB)
    agg = _sc_scatter(e.reshape(2, E, L), j4, jnp.zeros((NP, L), jnp.float32), NP)

    return _tc_node(x, agg[0, :N], agg[1, :N], Wn0, bn0, Wn1, bn1)


# speedup vs baseline: 9.1122x; 1.0839x over previous
import functools

import jax
import jax.numpy as jnp
from jax import lax
from jax.experimental import pallas as pl
from jax.experimental.pallas import tpu as pltpu
from jax.experimental.pallas import tpu_sc as plsc

NC = 2
NS = 16
SUB = 80
CG = SUB
KS2 = 1
CS = SUB * KS2


def _tc_tables(x, pos_posw, Wstack, Wpstack, bstack):
    N, L = x.shape
    Bn = 2000 if N % 2000 == 0 else N
    nb = N // Bn

    def body(x_ref, p_ref, w_ref, wp_ref, b_ref, o_ref):
        mm = jnp.dot(x_ref[...], w_ref[0], preferred_element_type=jnp.float32)
        mm = mm + jnp.dot(p_ref[...], wp_ref[0], preferred_element_type=jnp.float32)
        o_ref[...] = mm + b_ref[0]

    return pl.pallas_call(
        body,
        grid=(4, nb),
        in_specs=[
            pl.BlockSpec((Bn, L), lambda t, b: (b, 0)),
            pl.BlockSpec((Bn, 6), lambda t, b: (b, 0)),
            pl.BlockSpec((1, L, L), lambda t, b: (t, 0, 0)),
            pl.BlockSpec((1, 6, L), lambda t, b: (t, 0, 0)),
            pl.BlockSpec((1, 1, L), lambda t, b: (t, 0, 0)),
        ],
        out_specs=pl.BlockSpec((Bn, L), lambda t, b: (t * nb + b, 0)),
        out_shape=jax.ShapeDtypeStruct((4 * N, L), jnp.float32),
    )(x, pos_posw, Wstack, Wpstack, bstack.reshape(4, 1, L))


def _rsqrt16(x):
    xi = lax.bitcast_convert_type(x, jnp.int32)
    yi = jnp.int32(0x5F3759DF) - (xi >> 1)
    y = lax.bitcast_convert_type(yi, jnp.float32)
    for _ in range(3):
        y = y * (1.5 - 0.5 * x * y * y)
    return y


def _sc_gather(tab, ia3, ib3, pos_flat, wn, N):
    ntot = ia3.shape[0]
    L = tab.shape[1]
    EE = ntot * CG
    nchunk = ntot // (NC * NS)
    npos = pos_flat.shape[0]
    nsl = L // 16
    mesh = plsc.VectorSubcoreMesh(core_axis_name="c", subcore_axis_name="s")

    @functools.partial(
        pl.kernel,
        mesh=mesh,
        out_type=jax.ShapeDtypeStruct((EE, L), jnp.float32),
        scratch_types=[
            pltpu.VMEM((2, 1, SUB), jnp.int32),
            pltpu.VMEM((2, 1, SUB), jnp.int32),
            pltpu.VMEM((2, CG, L), jnp.float32),
            pltpu.VMEM((2, CG, L), jnp.float32),
            pltpu.VMEM((npos,), jnp.float32),
            pltpu.VMEM((4, L), jnp.float32),
            [pltpu.SemaphoreType.DMA] * 2,
            [pltpu.SemaphoreType.DMA] * 2,
            [pltpu.SemaphoreType.DMA] * 2,
        ],
        compiler_params=pltpu.CompilerParams(needs_layout_passes=False),
    )
    def gk(tab_h, ia_h, ib_h, pos_h, wn_h, out_h, ia_v, ib_v, ra_v, rb_v,
           pos_v, wn_v, si, sg, sw):
        wid = lax.axis_index("s") * NC + lax.axis_index("c")
        chunk0 = wid * nchunk
        pltpu.sync_copy(pos_h, pos_v)
        pltpu.sync_copy(wn_h, wn_v)
        is_main = wid < (NC * NS) // 2
        off_a = jnp.where(is_main, 0, 2 * N)
        off_b = jnp.where(is_main, N, 3 * N)
        toff = jnp.where(is_main, 0, 2)
        wnd = [wn_v[toff, pl.ds(k * 16, 16)] for k in range(nsl)]
        wndw = [wn_v[toff + 1, pl.ds(k * 16, 16)] for k in range(nsl)]

        def issue_idx(cidx, d):
            pltpu.async_copy(ia_h.at[cidx], ia_v.at[d], si[d])
            pltpu.async_copy(ib_h.at[cidx], ib_v.at[d], si[d])

        def wait_idx(d):
            pltpu.make_async_copy(ia_h.at[0], ia_v.at[d], si[d]).wait()
            pltpu.make_async_copy(ib_h.at[0], ib_v.at[d], si[d]).wait()

        def issue_gather(d):
            pltpu.async_copy(tab_h.at[ia_v.at[d, 0]], ra_v.at[d], sg[d])
            pltpu.async_copy(tab_h.at[ib_v.at[d, 0]], rb_v.at[d], sg[d])

        def wait_gather(d):
            pltpu.make_async_copy(tab_h.at[ia_v.at[d, 0]], ra_v.at[d], sg[d]).wait()
            pltpu.make_async_copy(tab_h.at[ib_v.at[d, 0]], rb_v.at[d], sg[d]).wait()

        def issue_write(cidx, d):
            pltpu.async_copy(ra_v.at[d], out_h.at[pl.ds(cidx * CG, CG)], sw[d])

        def wait_write(d):
            pltpu.make_async_copy(ra_v.at[d], out_h.at[pl.ds(0, CG)], sw[d]).wait()

        def compute(d):
            def group(l, cc):
                ia16 = ia_v[d, 0, pl.ds(l * 16, 16)] - off_a
                ib16 = ib_v[d, 0, pl.ds(l * 16, 16)] - off_b
                pa = ia16 * 6
                pb = ib16 * 6
                gi = [plsc.load_gather(pos_v, [pa + k]) for k in range(6)]
                gj = [plsc.load_gather(pos_v, [pb + k]) for k in range(6)]
                d0 = gi[0] - gj[0]
                d1 = gi[1] - gj[1]
                d2 = gi[2] - gj[2]
                w0 = gi[3] - gj[3]
                w1 = gi[4] - gj[4]
                w2 = gi[5] - gj[5]
                sd = d0 * d0 + d1 * d1 + d2 * d2
                sq = w0 * w0 + w1 * w1 + w2 * w2
                nd = sd * _rsqrt16(jnp.maximum(sd, 1e-30))
                ndw = sq * _rsqrt16(jnp.maximum(sq, 1e-30))
                for e in range(16):
                    cnd = nd[e]
                    cndw = ndw[e]
                    r = l * 16 + e
                    for k in range(nsl):
                        sl = pl.ds(k * 16, 16)
                        acc = ra_v[d, r, sl] + rb_v[d, r, sl]
                        ra_v[d, r, sl] = acc + cnd * wnd[k] + cndw * wndw[k]
                return cc

            lax.fori_loop(0, CG // 16, group, 0)

        issue_idx(chunk0, 0)
        wait_idx(0)
        issue_gather(0)
        issue_idx(chunk0 + 1, 1)

        def pair(k2, carry):
            for d in (0, 1):
                k = k2 * 2 + d
                nxt = 1 - d

                @pl.when(k + 1 < nchunk)
                def _():
                    wait_idx(nxt)

                    @pl.when(k > 0)
                    def _():
                        wait_write(nxt)

                    issue_gather(nxt)

                wait_gather(d)
                compute(d)
                issue_write(chunk0 + k, d)

                @pl.when(k + 2 < nchunk)
                def _():
                    issue_idx(chunk0 + k + 2, d)
            return carry

        lax.fori_loop(0, nchunk // 2, pair, 0)
        wait_write(0)
        wait_write(1)

    return gk(tab, ia3, ib3, pos_flat, wn)


def _tc_edge(S, W1s, b1s, E):
    EE = S.shape[0]
    Be = 4000 if E % 4000 == 0 else E
    nbe = E // Be

    def body(s_ref, w1_ref, b1_ref, o_ref):
        h = jnp.maximum(s_ref[...], 0.0).astype(jnp.bfloat16)
        h1 = jnp.dot(h, w1_ref[0], preferred_element_type=jnp.float32)
        h1 = h1 + b1_ref[0]
        mu = jnp.mean(h1, axis=1, keepdims=True)
        var = jnp.mean((h1 - mu) ** 2, axis=1, keepdims=True)
        o_ref[...] = (h1 - mu) * lax.rsqrt(var + 1e-5)

    return pl.pallas_call(
        body,
        grid=(EE // Be,),
        in_specs=[
            pl.BlockSpec((Be, 128), lambda t: (t, 0)),
            pl.BlockSpec((1, 128, 128), lambda t: (t // nbe, 0, 0)),
            pl.BlockSpec((1, 1, 128), lambda t: (t // nbe, 0, 0)),
        ],
        out_specs=pl.BlockSpec((Be, 128), lambda t: (t, 0)),
        out_shape=jax.ShapeDtypeStruct((EE, 128), jnp.float32),
    )(S, W1s, b1s.reshape(2, 1, 128))


def _sc_scatter(e2, j4, zeros, NP):
    _, E, L = e2.shape
    _, ntot, ks, _ = j4.shape
    C = ks * SUB
    nchunk = ntot // NS
    per_s = E // NS
    nrow = NP // NS
    mesh = plsc.VectorSubcoreMesh(core_axis_name="c", subcore_axis_name="s")

    @functools.partial(
        pl.kernel,
        mesh=mesh,
        out_type=jax.ShapeDtypeStruct((NC, NP, L), jnp.float32),
        scratch_types=[
            pltpu.VMEM((2, ks, SUB), jnp.int32),
            pltpu.VMEM((2, C, L), jnp.float32),
            pltpu.VMEM_SHARED((NP, L), jnp.float32),
            [pltpu.SemaphoreType.DMA] * 2,
            [pltpu.SemaphoreType.DMA] * 2,
        ],
    )
    def sk(e_h, j_h, z_h, out_h, j_v, rows_v, acc, sl, sa):
        c = lax.axis_index("c")
        s = lax.axis_index("s")
        pltpu.sync_copy(z_h.at[pl.ds(s * nrow, nrow)], acc.at[pl.ds(s * nrow, nrow)])
        plsc.subcore_barrier()

        def issue_load(k, d):
            cidx = s * nchunk + k
            pltpu.async_copy(j_h.at[c, cidx], j_v.at[d], sl[d])
            pltpu.async_copy(e_h.at[c, pl.ds(s * per_s + k * C, C)], rows_v.at[d], sl[d])

        def wait_load(d):
            pltpu.make_async_copy(j_h.at[c, 0], j_v.at[d], sl[d]).wait()
            pltpu.make_async_copy(e_h.at[c, pl.ds(0, C)], rows_v.at[d], sl[d]).wait()

        def issue_adds(d):
            for b in range(ks):
                src = rows_v.at[d, pl.ds(b * SUB, SUB)]
                pltpu.async_copy(src, acc.at[j_v.at[d, b]], sa[d], add=True)

        def wait_adds(d):
            for b in range(ks):
                src = rows_v.at[d, pl.ds(b * SUB, SUB)]
                pltpu.make_async_copy(src, acc.at[j_v.at[d, b]], sa[d]).wait()

        issue_load(0, 0)
        issue_load(1, 1)

        def pair(k2, carry):
            for d in (0, 1):
                k = k2 * 2 + d
                wait_load(d)
                issue_adds(d)

                @pl.when(k + 2 < nchunk)
                def _():
                    wait_adds(d)
                    issue_load(k + 2, d)
            return carry

        lax.fori_loop(0, nchunk // 2, pair, 0)
        wait_adds(0)
        wait_adds(1)
        plsc.subcore_barrier()
        pltpu.sync_copy(acc.at[pl.ds(s * nrow, nrow)], out_h.at[c, pl.ds(s * nrow, nrow)])

    return sk(e2, j4, zeros)


def _tc_node(x, am, ac, Wn0, bn0, Wn1, bn1):
    N, L = x.shape
    Bn = 2000 if N % 2000 == 0 else N

    def body(x_ref, a_ref, c_ref, w0_ref, b0_ref, w1_ref, b1_ref, o_ref):
        xb = x_ref[...]
        h = jnp.dot(xb, w0_ref[0:128, :], preferred_element_type=jnp.float32)
        h = h + jnp.dot(a_ref[...], w0_ref[128:256, :], preferred_element_type=jnp.float32)
        h = h + jnp.dot(c_ref[...], w0_ref[256:384, :], preferred_element_type=jnp.float32)
        h = jnp.maximum(h + b0_ref[0], 0.0)
        h1 = jnp.dot(h, w1_ref[...], preferred_element_type=jnp.float32)
        h1 = h1 + b1_ref[0]
        mu = jnp.mean(h1, axis=1, keepdims=True)
        var = jnp.mean((h1 - mu) ** 2, axis=1, keepdims=True)
        o_ref[...] = (h1 - mu) * lax.rsqrt(var + 1e-5) + xb

    return pl.pallas_call(
        body,
        grid=(N // Bn,),
        in_specs=[
            pl.BlockSpec((Bn, L), lambda b: (b, 0)),
            pl.BlockSpec((Bn, L), lambda b: (b, 0)),
            pl.BlockSpec((Bn, L), lambda b: (b, 0)),
            pl.BlockSpec((384, L), lambda b: (0, 0)),
            pl.BlockSpec((1, L), lambda b: (0, 0)),
            pl.BlockSpec((L, L), lambda b: (0, 0)),
            pl.BlockSpec((1, L), lambda b: (0, 0)),
        ],
        out_specs=pl.BlockSpec((Bn, L), lambda b: (b, 0)),
        out_shape=jax.ShapeDtypeStruct((N, L), jnp.float32),
    )(x, am, ac, Wn0, bn0.reshape(1, L), Wn1, bn1.reshape(1, L))


def kernel(x, g_cg, pos_posw, Wm0, bm0, Wm1, bm1, Wc0, bc0, Wc1, bc1,
           Wn0, bn0, Wn1, bn1):
    N, L = x.shape
    E = g_cg.shape[-1]
    z3 = jnp.zeros((3, L), jnp.float32)

    Wstack = jnp.stack([Wm0[8:136], Wm0[136:264], Wc0[4:132], Wc0[132:260]])
    Wp_m = jnp.concatenate([Wm0[0:3], Wm0[4:7]])
    Wp_c = jnp.concatenate([z3, Wc0[0:3]])
    Wpstack = jnp.stack([Wp_m, -Wp_m, Wp_c, -Wp_c])
    zb = jnp.zeros_like(bm0)
    bstack = jnp.stack([bm0, zb, bc0, zb])
    tab = _tc_tables(x, pos_posw, Wstack, Wpstack, bstack)

    wn = jnp.stack([Wm0[3], Wm0[7], jnp.zeros((L,), jnp.float32), Wc0[3]])

    iA = jnp.concatenate([g_cg[0, 0], g_cg[1, 0] + 2 * N])
    iB = jnp.concatenate([g_cg[0, 1] + N, g_cg[1, 1] + 3 * N])
    npos = ((6 * N + 127) // 128) * 128
    pos_flat = jnp.zeros((npos,), jnp.float32).at[: 6 * N].set(pos_posw.reshape(-1))
    S = _sc_gather(tab, iA.reshape(-1, 1, SUB), iB.reshape(-1, 1, SUB),
                   pos_flat, wn, N)

    W1s = jnp.stack([Wm1, Wc1]).astype(jnp.bfloat16)
    b1s = jnp.stack([bm1, bc1])
    e = _tc_edge(S, W1s, b1s, E)

    NP = ((N + 8 * NS - 1) // (8 * NS)) * (8 * NS)
    j4 = g_cg[:, 1, :].reshape(2, E // CS, KS2, SUB)
    agg = _sc_scatter(e.reshape(2, E, L), j4, jnp.zeros((NP, L), jnp.float32), NP)

    return _tc_node(x, agg[0, :N], agg[1, :N], Wn0, bn0, Wn1, bn1)
